# baseline (speedup 1.0000x reference)
import functools

import jax
import jax.numpy as jnp
from jax import lax
from jax.experimental import pallas as pl
from jax.experimental.pallas import tpu as pltpu
from jax.experimental.pallas import tpu_sc as plsc

B, S, D = 16, 2048, 768
W_MAX = 1024
NL = 9
LANES = 16
SC_LANES = 10

NC, NS = 2, 16
CHUNK = 2048


def _probs_body(x_ref, w_ref, wid_ref, o_ref):
    x = x_ref[0]
    w = w_ref[...]
    lt = lax.dot_general(w, x, (((0,), (1,)), ((), ())),
                         preferred_element_type=jnp.float32,
                         precision=lax.Precision.DEFAULT)
    m = jnp.max(lt, axis=0, keepdims=True)
    e = jnp.exp(lt - m)
    p = e / jnp.sum(e, axis=0, keepdims=True)
    o_ref[0, :NL, :] = p
    o_ref[0, NL:SC_LANES, :] = jnp.ones((1, CHUNK), jnp.float32)
    o_ref[0, SC_LANES:SC_LANES + 1, :] = lax.bitcast_convert_type(
        wid_ref[0], jnp.float32)


def _probs_call(x3, w_mlp, wid3):
    return pl.pallas_call(
        _probs_body,
        grid=(B * S // CHUNK,),
        in_specs=[
            pl.BlockSpec((1, CHUNK, D), lambda i: (i, 0, 0)),
            pl.BlockSpec((D, NL), lambda i: (0, 0)),
            pl.BlockSpec((1, 1, CHUNK), lambda i: (i, 0, 0)),
        ],
        out_specs=pl.BlockSpec((1, LANES, CHUNK), lambda i: (i, 0, 0)),
        out_shape=jax.ShapeDtypeStruct((B, LANES, S), jnp.float32),
        compiler_params=pltpu.CompilerParams(
            dimension_semantics=("arbitrary",)),
    )(x3, w_mlp, wid3)


def _seg_body(probsT_hbm, zeros_hbm, out_hbm, pt_v, acc_v, sem):
    c = lax.axis_index("c")
    s = lax.axis_index("s")
    b = c * NS + s

    @pl.when(b < B)
    def _():
        cp = pltpu.async_copy(probsT_hbm.at[b], pt_v, sem)
        cp3 = pltpu.async_copy(zeros_hbm, acc_v, sem)
        cp.wait()
        cp3.wait()

        def chunk(t, carry):
            for u in range(2):
                base = t * 32 + u * 16
                iw = plsc.bitcast(pt_v[SC_LANES, pl.ds(base, 16)], jnp.int32)
                for l in range(SC_LANES):
                    vals = pt_v[l, pl.ds(base, 16)]
                    plsc.addupdate_scatter(
                        acc_v, [iw + (l * W_MAX)], vals)
            return carry
        lax.fori_loop(0, S // 32, chunk, 0)

        pltpu.sync_copy(acc_v, out_hbm.at[b])


@functools.cache
def _seg_call():
    return pl.kernel(
        _seg_body,
        out_type=jax.ShapeDtypeStruct((B, SC_LANES * W_MAX), jnp.float32),
        mesh=plsc.VectorSubcoreMesh(core_axis_name="c", subcore_axis_name="s",
                                    num_cores=NC, num_subcores=NS),
        compiler_params=pltpu.CompilerParams(needs_layout_passes=False,
                                             use_tc_tiling_on_sc=False),
        scratch_types=[
            pltpu.VMEM((LANES, S), jnp.float32),
            pltpu.VMEM((SC_LANES * W_MAX,), jnp.float32),
            pltpu.SemaphoreType.DMA,
        ],
    )


def _final_body(sums_ref, labels_ref, la_ref, loss_ref):
    sums = sums_ref[...]
    lane = lax.broadcasted_iota(jnp.int32, sums.shape, 1)
    cnt = sums[:, NL:SC_LANES, :]
    avg = sums / jnp.maximum(cnt, 1.0)
    la_ref[...] = avg[:, :NL, :]
    valid = lane < NL
    e = jnp.where(valid, jnp.exp(avg), 0.0)
    lse = jnp.log(jnp.sum(e, axis=1))
    lab = labels_ref[...][:, None, :]
    picked = jnp.sum(jnp.where(lane == lab, avg, 0.0), axis=1)
    loss_ref[0, 0] = jnp.sum(lse - picked) / float(B * W_MAX)


def _final_call(sums3, labels):
    return pl.pallas_call(
        _final_body,
        in_specs=[
            pl.BlockSpec((B, SC_LANES, W_MAX), lambda: (0, 0, 0)),
            pl.BlockSpec((B, W_MAX), lambda: (0, 0)),
        ],
        out_specs=[
            pl.BlockSpec((B, NL, W_MAX), lambda: (0, 0, 0)),
            pl.BlockSpec(memory_space=pltpu.SMEM),
        ],
        out_shape=[
            jax.ShapeDtypeStruct((B, NL, W_MAX), jnp.float32),
            jax.ShapeDtypeStruct((1, 1), jnp.float32),
        ],
    )(sums3, labels)


def kernel(sent_logits, word_ids, labels, W_mlp):
    wid3 = word_ids.reshape(B, 1, S)
    probs_t = _probs_call(sent_logits, W_mlp, wid3)
    zeros = jnp.zeros((SC_LANES * W_MAX,), jnp.float32)
    sums = _seg_call()(probs_t, zeros)
    la_t, loss = _final_call(sums.reshape(B, SC_LANES, W_MAX), labels)
    return jnp.swapaxes(la_t, 1, 2), loss.reshape(())


# speedup vs baseline: 1.0330x; 1.0330x over previous
import functools

import jax
import jax.numpy as jnp
from jax import lax
from jax.experimental import pallas as pl
from jax.experimental.pallas import tpu as pltpu
from jax.experimental.pallas import tpu_sc as plsc

B, S, D = 16, 2048, 768
W_MAX = 1024
NL = 9
LANES = 16
SC_LANES = 10

NC, NS = 2, 16
CHUNK = 2048


def _probs_body(x_ref, w_ref, o_ref):
    x = x_ref[0]
    w = w_ref[...]
    lt = lax.dot_general(w, x, (((0,), (1,)), ((), ())),
                         preferred_element_type=jnp.float32,
                         precision=lax.Precision.DEFAULT)
    m = jnp.max(lt, axis=0, keepdims=True)
    e = jnp.exp(lt - m)
    p = e / jnp.sum(e, axis=0, keepdims=True)
    o_ref[0, :NL, :] = p
    o_ref[0, NL:SC_LANES, :] = jnp.ones((1, CHUNK), jnp.float32)


def _probs_call(x3, w_mlp):
    return pl.pallas_call(
        _probs_body,
        grid=(B * S // CHUNK,),
        in_specs=[
            pl.BlockSpec((1, CHUNK, D), lambda i: (i, 0, 0)),
            pl.BlockSpec((D, NL), lambda i: (0, 0)),
        ],
        out_specs=pl.BlockSpec((1, LANES, CHUNK), lambda i: (i, 0, 0)),
        out_shape=jax.ShapeDtypeStruct((B, LANES, S), jnp.float32),
        compiler_params=pltpu.CompilerParams(
            dimension_semantics=("arbitrary",)),
    )(x3, w_mlp)


def _seg_body(probsT_hbm, wids_hbm, zeros_hbm, out_hbm, idx_v, pt_v, acc_v, sem):
    c = lax.axis_index("c")
    s = lax.axis_index("s")
    b = c * NS + s

    @pl.when(b < B)
    def _():
        cp1 = pltpu.async_copy(wids_hbm.at[b], idx_v, sem)
        cp2 = pltpu.async_copy(probsT_hbm.at[b], pt_v, sem)
        cp3 = pltpu.async_copy(zeros_hbm, acc_v, sem)
        cp1.wait()
        cp2.wait()
        cp3.wait()

        def chunk(t, carry):
            for u in range(2):
                base = t * 32 + u * 16
                iw = idx_v[pl.ds(base, 16)]
                for l in range(SC_LANES):
                    vals = pt_v[l, pl.ds(base, 16)]
                    plsc.addupdate_scatter(
                        acc_v, [iw + (l * W_MAX)], vals)
            return carry
        lax.fori_loop(0, S // 32, chunk, 0)

        pltpu.sync_copy(acc_v, out_hbm.at[b])


@functools.cache
def _seg_call():
    return pl.kernel(
        _seg_body,
        out_type=jax.ShapeDtypeStruct((B, SC_LANES * W_MAX), jnp.float32),
        mesh=plsc.VectorSubcoreMesh(core_axis_name="c", subcore_axis_name="s",
                                    num_cores=NC, num_subcores=NS),
        compiler_params=pltpu.CompilerParams(needs_layout_passes=False,
                                             use_tc_tiling_on_sc=False),
        scratch_types=[
            pltpu.VMEM((S,), jnp.int32),
            pltpu.VMEM((LANES, S), jnp.float32),
            pltpu.VMEM((SC_LANES * W_MAX,), jnp.float32),
            pltpu.SemaphoreType.DMA,
        ],
    )


def _final_body(sums_ref, labels_ref, la_ref, loss_ref):
    sums = sums_ref[...]
    lane = lax.broadcasted_iota(jnp.int32, sums.shape, 1)
    cnt = sums[:, NL:SC_LANES, :]
    avg = sums / jnp.maximum(cnt, 1.0)
    la_ref[...] = avg[:, :NL, :]
    valid = lane < NL
    e = jnp.where(valid, jnp.exp(avg), 0.0)
    lse = jnp.log(jnp.sum(e, axis=1))
    lab = labels_ref[...][:, None, :]
    picked = jnp.sum(jnp.where(lane == lab, avg, 0.0), axis=1)
    loss_ref[0, 0] = jnp.sum(lse - picked) / float(B * W_MAX)


def _final_call(sums3, labels):
    return pl.pallas_call(
        _final_body,
        in_specs=[
            pl.BlockSpec((B, SC_LANES, W_MAX), lambda: (0, 0, 0)),
            pl.BlockSpec((B, W_MAX), lambda: (0, 0)),
        ],
        out_specs=[
            pl.BlockSpec((B, NL, W_MAX), lambda: (0, 0, 0)),
            pl.BlockSpec(memory_space=pltpu.SMEM),
        ],
        out_shape=[
            jax.ShapeDtypeStruct((B, NL, W_MAX), jnp.float32),
            jax.ShapeDtypeStruct((1, 1), jnp.float32),
        ],
    )(sums3, labels)


def kernel(sent_logits, word_ids, labels, W_mlp):
    probs_t = _probs_call(sent_logits, W_mlp)
    zeros = jnp.zeros((SC_LANES * W_MAX,), jnp.float32)
    sums = _seg_call()(probs_t, word_ids, zeros)
    la_t, loss = _final_call(sums.reshape(B, SC_LANES, W_MAX), labels)
    return jnp.swapaxes(la_t, 1, 2), loss.reshape(())


# speedup vs baseline: 1.0973x; 1.0623x over previous
import functools

import jax
import jax.numpy as jnp
from jax import lax
from jax.experimental import pallas as pl
from jax.experimental.pallas import tpu as pltpu
from jax.experimental.pallas import tpu_sc as plsc

B, S, D = 16, 2048, 768
W_MAX = 1024
NL = 9
LANES = 16
SC_LANES = 10

NC, NS = 2, 16
CHUNK = 2048


def _probs_body(x_ref, w_ref, o_ref):
    x = x_ref[0]
    w = w_ref[...]
    lt = lax.dot_general(w, x, (((0,), (1,)), ((), ())),
                         preferred_element_type=jnp.float32,
                         precision=lax.Precision.DEFAULT)
    m = jnp.max(lt, axis=0, keepdims=True)
    e = jnp.exp(lt - m)
    p = e / jnp.sum(e, axis=0, keepdims=True)
    o_ref[0, :NL, :] = p
    o_ref[0, NL:SC_LANES, :] = jnp.ones((1, CHUNK), jnp.float32)


def _probs_call(x3, w_mlp):
    return pl.pallas_call(
        _probs_body,
        grid=(B * S // CHUNK,),
        in_specs=[
            pl.BlockSpec((1, CHUNK, D), lambda i: (i, 0, 0)),
            pl.BlockSpec((D, NL), lambda i: (0, 0)),
        ],
        out_specs=pl.BlockSpec((1, LANES, CHUNK), lambda i: (i, 0, 0)),
        out_shape=jax.ShapeDtypeStruct((B, LANES, S), jnp.float32),
        compiler_params=pltpu.CompilerParams(
            dimension_semantics=("arbitrary",)),
    )(x3, w_mlp)


LH = SC_LANES // 2


def _seg_body(probsT_hbm, wids_hbm, zeros_hbm, out_hbm, idx_v, pt_v, acc_v, sem):
    c = lax.axis_index("c")
    s = lax.axis_index("s")
    wid = c * NS + s
    b = wid // 2
    h = wid % 2

    cp1 = pltpu.async_copy(wids_hbm.at[b], idx_v, sem)
    cp2 = pltpu.async_copy(probsT_hbm.at[b, pl.ds(h * LH, LH)], pt_v, sem)
    cp3 = pltpu.async_copy(zeros_hbm, acc_v, sem)
    cp1.wait()
    cp2.wait()
    cp3.wait()

    def chunk(t, carry):
        for u in range(2):
            base = t * 32 + u * 16
            iw = idx_v[pl.ds(base, 16)]
            for l in range(LH):
                vals = pt_v[l, pl.ds(base, 16)]
                plsc.addupdate_scatter(acc_v, [iw + (l * W_MAX)], vals)
        return carry
    lax.fori_loop(0, S // 32, chunk, 0)

    pltpu.sync_copy(acc_v, out_hbm.at[b, pl.ds(h * LH * W_MAX, LH * W_MAX)])


@functools.cache
def _seg_call():
    return pl.kernel(
        _seg_body,
        out_type=jax.ShapeDtypeStruct((B, SC_LANES * W_MAX), jnp.float32),
        mesh=plsc.VectorSubcoreMesh(core_axis_name="c", subcore_axis_name="s",
                                    num_cores=NC, num_subcores=NS),
        compiler_params=pltpu.CompilerParams(needs_layout_passes=False,
                                             use_tc_tiling_on_sc=False),
        scratch_types=[
            pltpu.VMEM((S,), jnp.int32),
            pltpu.VMEM((LH, S), jnp.float32),
            pltpu.VMEM((LH * W_MAX,), jnp.float32),
            pltpu.SemaphoreType.DMA,
        ],
    )


def _final_body(sums_ref, labels_ref, la_ref, loss_ref):
    sums = sums_ref[...]
    lane = lax.broadcasted_iota(jnp.int32, sums.shape, 1)
    cnt = sums[:, NL:SC_LANES, :]
    avg = sums / jnp.maximum(cnt, 1.0)
    la_ref[...] = avg[:, :NL, :]
    valid = lane < NL
    e = jnp.where(valid, jnp.exp(avg), 0.0)
    lse = jnp.log(jnp.sum(e, axis=1))
    lab = labels_ref[...][:, None, :]
    picked = jnp.sum(jnp.where(lane == lab, avg, 0.0), axis=1)
    loss_ref[0, 0] = jnp.sum(lse - picked) / float(B * W_MAX)


def _final_call(sums3, labels):
    return pl.pallas_call(
        _final_body,
        in_specs=[
            pl.BlockSpec((B, SC_LANES, W_MAX), lambda: (0, 0, 0)),
            pl.BlockSpec((B, W_MAX), lambda: (0, 0)),
        ],
        out_specs=[
            pl.BlockSpec((B, NL, W_MAX), lambda: (0, 0, 0)),
            pl.BlockSpec(memory_space=pltpu.SMEM),
        ],
        out_shape=[
            jax.ShapeDtypeStruct((B, NL, W_MAX), jnp.float32),
            jax.ShapeDtypeStruct((1, 1), jnp.float32),
        ],
    )(sums3, labels)


def kernel(sent_logits, word_ids, labels, W_mlp):
    probs_t = _probs_call(sent_logits, W_mlp)
    zeros = jnp.zeros((LH * W_MAX,), jnp.float32)
    sums = _seg_call()(probs_t, word_ids, zeros)
    la_t, loss = _final_call(sums.reshape(B, SC_LANES, W_MAX), labels)
    return jnp.swapaxes(la_t, 1, 2), loss.reshape(())


# speedup vs baseline: 1.1837x; 1.0787x over previous
import functools

import jax
import jax.numpy as jnp
from jax import lax
from jax.experimental import pallas as pl
from jax.experimental.pallas import tpu as pltpu
from jax.experimental.pallas import tpu_sc as plsc

B, S, D = 16, 2048, 768
W_MAX = 1024
NL = 9
LANES = 16
SC_LANES = 10
LH = SC_LANES // 2

NC, NS = 2, 16
CHUNK = 2048
ST = S // 128
WT = W_MAX // 128


def _probs_body(x_ref, w_ref, o_ref):
    x = x_ref[0]
    w = w_ref[...]
    lt = lax.dot_general(w, x, (((0,), (1,)), ((), ())),
                         preferred_element_type=jnp.float32,
                         precision=lax.Precision.DEFAULT)
    m = jnp.max(lt, axis=0, keepdims=True)
    e = jnp.exp(lt - m)
    p = e / jnp.sum(e, axis=0, keepdims=True)
    p4 = p.reshape(NL, ST, 128)
    o_ref[0, :NL] = p4
    o_ref[0, NL:SC_LANES] = jnp.ones((1, ST, 128), jnp.float32)


def _probs_call(x3, w_mlp):
    return pl.pallas_call(
        _probs_body,
        grid=(B * S // CHUNK,),
        in_specs=[
            pl.BlockSpec((1, CHUNK, D), lambda i: (i, 0, 0)),
            pl.BlockSpec((D, NL), lambda i: (0, 0)),
        ],
        out_specs=pl.BlockSpec((1, LANES, ST, 128), lambda i: (i, 0, 0, 0)),
        out_shape=jax.ShapeDtypeStruct((B, LANES, ST, 128), jnp.float32),
        compiler_params=pltpu.CompilerParams(
            dimension_semantics=("arbitrary",)),
    )(x3, w_mlp)


def _seg_body(probsT_hbm, wids_hbm, zeros_hbm, out_hbm, idx_v, pt_v, acc_v, sem):
    c = lax.axis_index("c")
    s = lax.axis_index("s")
    wid = c * NS + s
    b = wid // 2
    h = wid % 2

    cp1 = pltpu.async_copy(wids_hbm.at[b], idx_v, sem)
    cp2 = pltpu.async_copy(probsT_hbm.at[b, pl.ds(h * LH, LH)], pt_v, sem)
    cp3 = pltpu.async_copy(zeros_hbm, acc_v, sem)
    cp1.wait()
    cp2.wait()
    cp3.wait()

    def chunk(j, carry):
        for k in range(8):
            iw = idx_v[j, pl.ds(k * 16, 16)]
            iw_hi = lax.shift_right_logical(iw, 7)
            iw_lo = lax.bitwise_and(iw, 127)
            for l in range(LH):
                vals = pt_v[l, j, pl.ds(k * 16, 16)]
                plsc.addupdate_scatter(
                    acc_v, [jnp.full((16,), l, jnp.int32), iw_hi, iw_lo],
                    vals)
        return carry
    lax.fori_loop(0, ST, chunk, 0)

    pltpu.sync_copy(acc_v, out_hbm.at[b, pl.ds(h * LH, LH)])


@functools.cache
def _seg_call():
    return pl.kernel(
        _seg_body,
        out_type=jax.ShapeDtypeStruct((B, SC_LANES, WT, 128), jnp.float32),
        mesh=plsc.VectorSubcoreMesh(core_axis_name="c", subcore_axis_name="s",
                                    num_cores=NC, num_subcores=NS),
        compiler_params=pltpu.CompilerParams(needs_layout_passes=False,
                                             use_tc_tiling_on_sc=False),
        scratch_types=[
            pltpu.VMEM((ST, 128), jnp.int32),
            pltpu.VMEM((LH, ST, 128), jnp.float32),
            pltpu.VMEM((LH, WT, 128), jnp.float32),
            pltpu.SemaphoreType.DMA,
        ],
    )


def _final_body(sums_ref, labels_ref, la_ref, loss_ref):
    sums = sums_ref[...]
    lane = lax.broadcasted_iota(jnp.int32, sums.shape, 1)
    cnt = sums[:, NL:SC_LANES]
    avg = sums / jnp.maximum(cnt, 1.0)
    la_ref[...] = avg[:, :NL]
    valid = lane < NL
    e = jnp.where(valid, jnp.exp(avg), 0.0)
    lse = jnp.log(jnp.sum(e, axis=1))
    lab = labels_ref[...][:, None]
    picked = jnp.sum(jnp.where(lane == lab, avg, 0.0), axis=1)
    loss_ref[0, 0] = jnp.sum(lse - picked) / float(B * W_MAX)


def _final_call(sums4, labels3):
    return pl.pallas_call(
        _final_body,
        in_specs=[
            pl.BlockSpec((B, SC_LANES, WT, 128), lambda: (0, 0, 0, 0)),
            pl.BlockSpec((B, WT, 128), lambda: (0, 0, 0)),
        ],
        out_specs=[
            pl.BlockSpec((B, NL, WT, 128), lambda: (0, 0, 0, 0)),
            pl.BlockSpec(memory_space=pltpu.SMEM),
        ],
        out_shape=[
            jax.ShapeDtypeStruct((B, NL, WT, 128), jnp.float32),
            jax.ShapeDtypeStruct((1, 1), jnp.float32),
        ],
    )(sums4, labels3)


def kernel(sent_logits, word_ids, labels, W_mlp):
    probs_t = _probs_call(sent_logits, W_mlp)
    zeros = jnp.zeros((LH, WT, 128), jnp.float32)
    sums = _seg_call()(probs_t, word_ids.reshape(B, ST, 128),
                       zeros)
    la4, loss = _final_call(sums, labels.reshape(B, WT, 128))
    la = la4.reshape(B, NL, W_MAX)
    return jnp.swapaxes(la, 1, 2), loss.reshape(())
